# baseline (speedup 1.0000x reference)
import functools

import numpy as np
import jax
import jax.numpy as jnp
from jax import lax
from jax.experimental import pallas as pl
from jax.experimental.pallas import tpu as pltpu
from jax.experimental.pallas import tpu_sc as plsc

N_B = 10000
E_B = 160000
N_C = 5000
E_C = 80000

CHUNK = 2048
BN = 256
SC_ALIGN = 6144
SC_CG = 96


def _ceil_to(x, m):
    return (x + m - 1) // m * m


def _mm2_body(x_ref, wl_ref, wr_ref, o1_ref, o2_ref):
    x = x_ref[...]
    o1_ref[...] = jnp.dot(x, wl_ref[...], preferred_element_type=jnp.float32)
    o2_ref[...] = jnp.dot(x, wr_ref[...], preferred_element_type=jnp.float32)


def _mm2(x, Wl, Wr):
    npad, D = x.shape
    W = Wl.shape[1]
    RB = npad // 8
    out = jax.ShapeDtypeStruct((npad, W), jnp.float32)
    o1, o2 = pl.pallas_call(
        _mm2_body,
        grid=(8,),
        in_specs=[
            pl.BlockSpec((RB, D), lambda i: (i, 0)),
            pl.BlockSpec((D, W), lambda i: (0, 0)),
            pl.BlockSpec((D, W), lambda i: (0, 0)),
        ],
        out_specs=[
            pl.BlockSpec((RB, W), lambda i: (i, 0)),
            pl.BlockSpec((RB, W), lambda i: (i, 0)),
        ],
        out_shape=[out, out],
    )(x, Wl, Wr)
    return o1, o2


def _sc_gather_body(n_iters, table_hbm, idx_hbm, out_hbm,
                    idx_v, buf0, buf1, sem0, sem1):
    nc = 2
    wid = lax.axis_index("s") * nc + lax.axis_index("c")
    per_w = n_iters * SC_CG
    base = wid * per_w
    pltpu.sync_copy(idx_hbm.at[pl.ds(base, per_w)], idx_v)
    bufs = (buf0, buf1)
    sems = (sem0, sem1)

    pltpu.async_copy(table_hbm.at[idx_v.at[pl.ds(0, SC_CG)]], buf0, sem0)

    def body(i2, carry):
        for b in (0, 1):
            i = i2 * 2 + b
            nxt = bufs[1 - b]
            nsem = sems[1 - b]

            @pl.when(i + 1 < n_iters)
            def _():
                pltpu.async_copy(
                    table_hbm.at[idx_v.at[pl.ds((i + 1) * SC_CG, SC_CG)]],
                    nxt, nsem)

            pltpu.make_async_copy(
                table_hbm.at[idx_v.at[pl.ds(i * SC_CG, SC_CG)]],
                bufs[b], sems[b]).wait()
            pltpu.sync_copy(bufs[b], out_hbm.at[pl.ds(base + i * SC_CG, SC_CG)])
        return carry

    lax.fori_loop(0, n_iters // 2, body, 0)


def _sc_gather(table, idx):
    B = idx.shape[0]
    D = table.shape[1]
    assert B % SC_ALIGN == 0 and D % 16 == 0
    n_iters = B // 32 // SC_CG
    assert n_iters % 2 == 0
    mesh = plsc.VectorSubcoreMesh(core_axis_name="c", subcore_axis_name="s")
    fn = pl.kernel(
        functools.partial(_sc_gather_body, n_iters),
        out_type=jax.ShapeDtypeStruct((B, D), jnp.float32),
        mesh=mesh,
        scratch_types=[
            pltpu.VMEM((n_iters * SC_CG,), jnp.int32),
            pltpu.VMEM((SC_CG, D), jnp.float32),
            pltpu.VMEM((SC_CG, D), jnp.float32),
            pltpu.SemaphoreType.DMA,
            pltpu.SemaphoreType.DMA,
        ],
    )
    return fn(table, idx)


def _conv_body(relu, logsm,
               cof_ref, blk_ref, first_ref, last_ref,
               gl_ref, dst_ref, xr_ref, a_ref, a2_ref, bias_ref,
               out_ref, acc_ref, psum_ref):
    g = pl.program_id(0)
    blk = blk_ref[g]
    first = first_ref[g]
    last = last_ref[g]

    @pl.when(first == 1)
    def _():
        acc_ref[...] = jnp.zeros_like(acc_ref)
        psum_ref[...] = jnp.zeros_like(psum_ref)

    dstv = dst_ref[0, 0, :]
    local = dstv - blk * BN
    iota_c = lax.broadcasted_iota(jnp.int32, (CHUNK, BN), 1)
    iota_b = lax.broadcasted_iota(jnp.int32, (BN, CHUNK), 0)
    T = (local[:, None] == iota_c).astype(jnp.float32)
    S = (local[None, :] == iota_b).astype(jnp.float32)

    gl = gl_ref[...]
    gr = jnp.dot(T, xr_ref[...], preferred_element_type=jnp.float32)
    e = gl + gr
    e = jnp.where(e >= 0.0, e, 0.2 * e)
    alpha = jnp.dot(e, a_ref[...], preferred_element_type=jnp.float32)
    p = jnp.exp(alpha)
    pw = jnp.dot(p, a2_ref[...], preferred_element_type=jnp.float32)
    msg = gl * pw
    acc_ref[...] += jnp.dot(S, msg, preferred_element_type=jnp.float32)
    psum_ref[...] += jnp.dot(S, p, preferred_element_type=jnp.float32)

    @pl.when(last == 1)
    def _():
        denom = jnp.dot(psum_ref[...], a2_ref[...],
                        preferred_element_type=jnp.float32) + 1e-16
        res = acc_ref[...] / denom + bias_ref[0, :][None, :]
        if relu:
            res = jnp.maximum(res, 0.0)
        if logsm:
            c0 = res[:, 0:1]
            c1 = res[:, 1:2]
            m = jnp.maximum(c0, c1)
            lse = m + jnp.log(jnp.exp(c0 - m) + jnp.exp(c1 - m))
            res = res - lse
        out_ref[...] = res


def _conv(gl, dst3, xr, A, A2, bias, sched, relu, logsm):
    npad, W = xr.shape
    L = sched["cof"].shape[0]
    grid_spec = pltpu.PrefetchScalarGridSpec(
        num_scalar_prefetch=4,
        grid=(L,),
        in_specs=[
            pl.BlockSpec((CHUNK, W), lambda g, cof, blk, fi, la: (cof[g], 0)),
            pl.BlockSpec((1, 1, CHUNK), lambda g, cof, blk, fi, la: (cof[g], 0, 0)),
            pl.BlockSpec((BN, W), lambda g, cof, blk, fi, la: (blk[g], 0)),
            pl.BlockSpec((A.shape[0], 8), lambda g, cof, blk, fi, la: (0, 0)),
            pl.BlockSpec((8, W), lambda g, cof, blk, fi, la: (0, 0)),
            pl.BlockSpec((1, W), lambda g, cof, blk, fi, la: (0, 0)),
        ],
        out_specs=pl.BlockSpec((BN, W), lambda g, cof, blk, fi, la: (blk[g], 0)),
        scratch_shapes=[
            pltpu.VMEM((BN, W), jnp.float32),
            pltpu.VMEM((BN, 8), jnp.float32),
        ],
    )
    return pl.pallas_call(
        functools.partial(_conv_body, relu, logsm),
        grid_spec=grid_spec,
        out_shape=jax.ShapeDtypeStruct((npad, W), jnp.float32),
    )(sched["cof"], sched["blk"], sched["first"], sched["last"],
      gl, dst3, xr, A, A2, bias)


def _fa_body(x1_ref, x2_ref, w1_ref, b1_ref, w2_ref, b2_ref, o_ref):
    x1 = x1_ref[...]
    x2 = x2_ref[...]
    cb = jnp.concatenate([x1, x2], axis=1)
    acc0 = jnp.zeros((x1.shape[0], 1), jnp.float32)
    acc1 = jnp.zeros((x1.shape[0], 1), jnp.float32)
    for h in range(8):
        hh = jnp.dot(cb, w1_ref[h], preferred_element_type=jnp.float32)
        hh = jnp.maximum(hh + b1_ref[h][None, :], 0.0)
        lg = jnp.dot(hh, w2_ref[h], preferred_element_type=jnp.float32)
        lg = lg + b2_ref[h][None, :]
        l0 = lg[:, 0:1]
        l1 = lg[:, 1:2]
        m = jnp.maximum(l0, l1)
        e0 = jnp.exp(l0 - m)
        e1 = jnp.exp(l1 - m)
        z = e0 + e1
        acc0 += e0 / z
        acc1 += e1 / z
    o_ref[...] = jnp.concatenate([x1 * (acc0 / 8.0), x2 * (acc1 / 8.0)], axis=1)


def _fa(x1, x2, W1, b1, W2, b2):
    npad = x1.shape[0]
    RB = npad // 8
    return pl.pallas_call(
        _fa_body,
        grid=(8,),
        in_specs=[
            pl.BlockSpec((RB, 128), lambda i: (i, 0)),
            pl.BlockSpec((RB, 128), lambda i: (i, 0)),
            pl.BlockSpec((8, 256, 128), lambda i: (0, 0, 0)),
            pl.BlockSpec((8, 128), lambda i: (0, 0)),
            pl.BlockSpec((8, 128, 2), lambda i: (0, 0, 0)),
            pl.BlockSpec((8, 2), lambda i: (0, 0)),
        ],
        out_specs=pl.BlockSpec((RB, 256), lambda i: (i, 0)),
        out_shape=jax.ShapeDtypeStruct((npad, 256), jnp.float32),
    )(x1, x2, W1, b1, W2, b2)


def _schedule(edge_index, n, npad):
    etot = edge_index.shape[1] + n
    sl = jnp.arange(n, dtype=edge_index.dtype)
    src = jnp.concatenate([edge_index[0], sl])
    dst = jnp.concatenate([edge_index[1], sl])
    perm = jnp.argsort(dst)
    src_s = src[perm]
    dst_s = dst[perm]

    nchunk = -(-etot // CHUNK)
    ep_conv = nchunk * CHUNK
    ep_gather = _ceil_to(etot, SC_ALIGN)
    nblk = npad // BN

    src_p = jnp.pad(src_s, (0, ep_gather - etot))
    dst_p = jnp.pad(dst_s, (0, ep_conv - etot), constant_values=-1)
    dst3 = dst_p.reshape(nchunk, 1, CHUNK)

    first_idx = jnp.arange(nchunk, dtype=jnp.int32) * CHUNK
    last_idx = jnp.minimum(first_idx + CHUNK, etot) - 1
    lo = dst_s[first_idx] // BN
    hi = dst_s[last_idx] // BN
    s = hi - lo + 1
    cum = jnp.cumsum(s)
    V = cum[-1]

    L = nchunk + nblk
    t = jnp.arange(L, dtype=jnp.int32)
    cof = jnp.searchsorted(cum, t, side="right").astype(jnp.int32)
    cof = jnp.minimum(cof, nchunk - 1)
    start = cum[cof] - s[cof]
    blk = lo[cof] + (t - start)
    valid = t < V
    blk = jnp.where(valid, blk, hi[-1]).astype(jnp.int32)
    prev_blk = jnp.concatenate([jnp.array([-1], jnp.int32), blk[:-1]])
    nxt_blk = jnp.concatenate([blk[1:], jnp.array([-2], jnp.int32)])
    nxt_valid = jnp.concatenate([valid[1:], jnp.array([False])])
    first = (valid & (blk != prev_blk)).astype(jnp.int32)
    last = (valid & ((~nxt_valid) | (blk != nxt_blk))).astype(jnp.int32)
    return {"src": src_p, "dst3": dst3, "cof": cof, "blk": blk,
            "first": first, "last": last}


def _expand_att(att, W):
    H, C = att.shape
    eye = jnp.eye(8, dtype=jnp.float32)
    A = (att[:, :, None] * eye[:H][:, None, :]).reshape(H * C, 8)
    A = jnp.pad(A, ((0, W - H * C), (0, 0)))
    head_of = jnp.arange(H * C, dtype=jnp.int32) // C
    A2 = (jnp.arange(8, dtype=jnp.int32)[:, None] == head_of[None, :])
    A2 = jnp.pad(A2.astype(jnp.float32), ((0, 0), (0, W - H * C)))
    return A, A2


def _gatv2_layer(x, Wl, Wr, att, bias, sched, relu=True, logsm=False):
    W = Wl.shape[1]
    xl, xr = _mm2(x, Wl, Wr)
    gl = _sc_gather(xl, sched["src"])
    A, A2 = _expand_att(att, W)
    b2d = bias[None, :]
    return _conv(gl, sched["dst3"], xr, A, A2, b2d, sched, relu, logsm)


def kernel(building_features, community_features, c1_Wl, c1_Wr, c1_att, c1_b, c2_Wl, c2_Wr, c2_att, c2_b, fa_W1, fa_b1, fa_W2, fa_b2, b1_Wl, b1_Wr, b1_att, b1_b, b2_Wl, b2_Wr, b2_att, b2_b, b3_Wl, b3_Wr, b3_att, b3_b, building_edge_index, community_edge_index, building_to_comm_mapping):
    npc = _ceil_to(N_C, 512)
    npb = _ceil_to(N_B, 512)

    sch_c = _schedule(community_edge_index, N_C, npc)
    sch_b = _schedule(building_edge_index, N_B, npb)

    cf = jnp.pad(community_features, ((0, npc - N_C), (0, 0)))
    bf = jnp.pad(building_features, ((0, npb - N_B), (0, 0)))

    cx = _gatv2_layer(cf, c1_Wl, c1_Wr, c1_att, c1_b, sch_c)
    cx = _gatv2_layer(cx, c2_Wl, c2_Wr, c2_att, c2_b, sch_c)

    bmap = jnp.pad(building_to_comm_mapping,
                   (0, _ceil_to(N_B, SC_ALIGN) - N_B))
    bcf = _sc_gather(cx, bmap)[:npb]

    fused = _fa(bf, bcf, fa_W1, fa_b1, fa_W2, fa_b2)

    x = _gatv2_layer(fused, b1_Wl, b1_Wr, b1_att, b1_b, sch_b)
    x = _gatv2_layer(x, b2_Wl, b2_Wr, b2_att, b2_b, sch_b)
    b3_Wl_p = jnp.pad(b3_Wl, ((0, 0), (0, 126)))
    b3_Wr_p = jnp.pad(b3_Wr, ((0, 0), (0, 126)))
    b3_b_p = jnp.pad(b3_b, (0, 126))
    x = _gatv2_layer(x, b3_Wl_p, b3_Wr_p, b3_att, b3_b_p, sch_b,
                     relu=False, logsm=True)
    return x[:N_B, :2]
